# baseline (speedup 1.0000x reference)
import functools

import jax
import jax.numpy as jnp
from jax import lax
from jax.experimental import pallas as pl
from jax.experimental.pallas import tpu as pltpu
from jax.experimental.pallas import tpu_sc as plsc

B = 1024
S = 16
A = 100000
ABLK = 8192
NBLK = (A + ABLK - 1) // ABLK
NPAD = NBLK * ABLK
NSLC = ABLK // 128

_SC = plsc.get_sparse_core_info()
NC, NS = _SC.num_cores, _SC.num_subcores
NW = NC * NS
CH = B // NW
NIDX = S * CH
NQ = NIDX // 128


def _ac_body(state_ref, wpa_ref, wv_ref,
             value_ref, logz_ref, ent_ref,
             mvec_ref, m_ref, svec_ref, tvec_ref):
    p = pl.program_id(0)
    j = pl.program_id(1)

    @pl.when((p == 0) & (j == 0))
    def _init():
        mvec_ref[...] = jnp.full((8, 128), -1e30, jnp.float32)
        svec_ref[...] = jnp.zeros((B, 1), jnp.float32)
        tvec_ref[...] = jnp.zeros((B, 1), jnp.float32)
        value_ref[...] = jnp.dot(state_ref[...], wv_ref[...],
                                 preferred_element_type=jnp.float32)

    @pl.when(p == 0)
    def _bound_pass():
        w = wpa_ref[0:S, :]
        nsq = jnp.sum(w * w, axis=0, keepdims=True)
        bprow = wpa_ref[S:S + 1, :]
        nv = mvec_ref[0:1, :]
        bpv = mvec_ref[1:2, :]
        for k in range(NSLC):
            nv = jnp.maximum(nv, nsq[:, k * 128:(k + 1) * 128])
            bpv = jnp.maximum(bpv, bprow[:, k * 128:(k + 1) * 128])
        mvec_ref[0:1, :] = nv
        mvec_ref[1:2, :] = bpv

        @pl.when(j == NBLK - 1)
        def _finish_bound():
            wmax = jnp.sqrt(jnp.max(mvec_ref[0:1, :]))
            bpmax = jnp.max(mvec_ref[1:2, :])
            st = state_ref[...]
            snorm = jnp.sqrt(jnp.maximum(
                jnp.sum(st * st, axis=1, keepdims=True) - 1.0, 0.0))
            m_ref[...] = snorm * wmax + bpmax

    @pl.when(p == 1)
    def _exp_pass():
        lhs = jnp.concatenate(
            [state_ref[:, 0:S + 1], -m_ref[...]], axis=1)
        l2 = jnp.dot(lhs, wpa_ref[...],
                     preferred_element_type=jnp.float32)
        onescol = jnp.ones((2048, 1), jnp.float32)
        for k in range(ABLK // 2048):
            lk = l2[:, k * 2048:(k + 1) * 2048]
            ek = jnp.exp(lk)
            elk = ek * lk
            svec_ref[...] += jnp.dot(ek, onescol,
                                     preferred_element_type=jnp.float32)
            tvec_ref[...] += jnp.dot(elk, onescol,
                                     preferred_element_type=jnp.float32)

        @pl.when(j == NBLK - 1)
        def _fin():
            s = svec_ref[...]
            t2 = tvec_ref[...]
            logs = jnp.log(s)
            logz_ref[...] = m_ref[...] + logs
            ent_ref[...] = logs - t2 / s


@functools.partial(
    pl.kernel,
    mesh=plsc.VectorSubcoreMesh(core_axis_name="c", subcore_axis_name="s"),
    out_type=jax.ShapeDtypeStruct((B,), jnp.float32),
    scratch_types=[
        pltpu.VMEM((CH,), jnp.int32),
        pltpu.VMEM((NQ, 128), jnp.int32),
        pltpu.VMEM((NQ, 128), jnp.float32),
        pltpu.VMEM((CH,), jnp.float32),
        pltpu.VMEM((S, CH), jnp.float32),
        pltpu.VMEM((CH,), jnp.float32),
        pltpu.SemaphoreType.DMA,
    ],
)
def _sc_gather(wp_flat, bp_hbm, st3_hbm, act_hbm, la_hbm,
               a_v, idx_v, g_v, bpa_v, st_v, out_v, sem):
    wid = lax.axis_index("s") * NC + lax.axis_index("c")
    base = wid * CH
    pltpu.sync_copy(act_hbm.at[pl.ds(base, CH)], a_v)
    pltpu.sync_copy(st3_hbm.at[wid], st_v)
    pltpu.async_copy(bp_hbm.at[a_v], bpa_v, sem).wait()
    for k in range(S):
        for c in range(CH // 16):
            lin = k * CH + c * 16
            q, off = lin // 128, lin % 128
            av = a_v[pl.ds(c * 16, 16)]
            idx_v[q, pl.ds(off, 16)] = av + k * A
    copies = [
        pltpu.async_copy(wp_flat.at[idx_v.at[q]], g_v.at[q], sem)
        for q in range(NQ)
    ]
    for cp in copies:
        cp.wait()
    for c in range(CH // 16):
        acc = bpa_v[pl.ds(c * 16, 16)]
        for k in range(S):
            lin = k * CH + c * 16
            q, off = lin // 128, lin % 128
            acc = acc + g_v[q, pl.ds(off, 16)] * st_v[k, pl.ds(c * 16, 16)]
        out_v[pl.ds(c * 16, 16)] = acc
    pltpu.sync_copy(out_v, la_hbm.at[pl.ds(base, CH)])


@jax.jit
def _ac_call(state, action, Wp, bp, Wv, bv):
    bp_p = jnp.pad(bp, (0, NPAD - A), constant_values=-1e30)
    wpa = jnp.concatenate(
        [jnp.pad(Wp, ((0, 0), (0, NPAD - A))),
         bp_p[None, :],
         jnp.ones((1, NPAD), jnp.float32)], axis=0)
    state_aug = jnp.concatenate(
        [state, jnp.ones((B, 1), jnp.float32),
         jnp.zeros((B, 1), jnp.float32)], axis=1)
    wv_aug = jnp.concatenate(
        [Wv, bv[None, :], jnp.zeros((1, 1), jnp.float32)], axis=0)
    act_i = action.astype(jnp.int32)
    wp_flat = Wp.reshape(S * A)
    st3 = state.T.reshape(S, NW, CH).transpose(1, 0, 2)

    la = _sc_gather(wp_flat, bp, st3, act_i)

    value, logz, ent = pl.pallas_call(
        _ac_body,
        grid=(2, NBLK),
        in_specs=[
            pl.BlockSpec((B, S + 2), lambda p, j: (0, 0)),
            pl.BlockSpec((S + 2, ABLK), lambda p, j: (0, j)),
            pl.BlockSpec((S + 2, 1), lambda p, j: (0, 0)),
        ],
        out_specs=[
            pl.BlockSpec((B, 1), lambda p, j: (0, 0)),
            pl.BlockSpec((B, 1), lambda p, j: (0, 0)),
            pl.BlockSpec((B, 1), lambda p, j: (0, 0)),
        ],
        out_shape=[
            jax.ShapeDtypeStruct((B, 1), jnp.float32),
            jax.ShapeDtypeStruct((B, 1), jnp.float32),
            jax.ShapeDtypeStruct((B, 1), jnp.float32),
        ],
        scratch_shapes=[
            pltpu.VMEM((8, 128), jnp.float32),
            pltpu.VMEM((B, 1), jnp.float32),
            pltpu.VMEM((B, 1), jnp.float32),
            pltpu.VMEM((B, 1), jnp.float32),
        ],
    )(state_aug, wpa, wv_aug)
    return value, la - logz[:, 0], ent[:, 0]


def kernel(state, action, Wp, bp, Wv, bv):
    return _ac_call(state, action, Wp, bp, Wv, bv)


# speedup vs baseline: 1.6216x; 1.6216x over previous
import functools

import jax
import jax.numpy as jnp
from jax import lax
from jax.experimental import pallas as pl
from jax.experimental.pallas import tpu as pltpu
from jax.experimental.pallas import tpu_sc as plsc

B = 1024
S = 16
A = 100000
ABLK = 8192
NBLK = (A + ABLK - 1) // ABLK
NPAD = NBLK * ABLK
NSLC = ABLK // 128

_SC = plsc.get_sparse_core_info()
NC, NS = _SC.num_cores, _SC.num_subcores
NW = NC * NS
CH = B // NW
NIDX = S * CH
NQ = NIDX // 128


def _ac_body(state_ref, wpa_ref, wv_ref,
             value_ref, logz_ref, ent_ref,
             mvec_ref, m_ref, svec_ref, tvec_ref):
    p = pl.program_id(0)
    j = pl.program_id(1)

    @pl.when((p == 0) & (j == 0))
    def _init():
        mvec_ref[...] = jnp.full((8, 128), -1e30, jnp.float32)
        svec_ref[...] = jnp.zeros((B, 128), jnp.float32)
        tvec_ref[...] = jnp.zeros((B, 128), jnp.float32)
        value_ref[...] = jnp.dot(state_ref[...], wv_ref[...],
                                 preferred_element_type=jnp.float32)

    @pl.when(p == 0)
    def _bound_pass():
        w = wpa_ref[0:S, :]
        nsq = jnp.sum(w * w, axis=0, keepdims=True)
        bprow = wpa_ref[S:S + 1, :]
        nv = mvec_ref[0:1, :]
        bpv = mvec_ref[1:2, :]
        for k in range(NSLC):
            nv = jnp.maximum(nv, nsq[:, k * 128:(k + 1) * 128])
            bpv = jnp.maximum(bpv, bprow[:, k * 128:(k + 1) * 128])
        mvec_ref[0:1, :] = nv
        mvec_ref[1:2, :] = bpv

        @pl.when(j == NBLK - 1)
        def _finish_bound():
            wmax = jnp.sqrt(jnp.max(mvec_ref[0:1, :]))
            bpmax = jnp.max(mvec_ref[1:2, :])
            st = state_ref[...]
            snorm = jnp.sqrt(jnp.maximum(
                jnp.sum(st * st, axis=1, keepdims=True) - 1.0, 0.0))
            m_ref[...] = snorm * wmax + bpmax

    @pl.when(p == 1)
    def _exp_pass():
        lhs = jnp.concatenate(
            [state_ref[:, 0:S + 1], -m_ref[...]], axis=1)
        l2 = jnp.dot(lhs, wpa_ref[...],
                     preferred_element_type=jnp.float32)
        e = jnp.exp(l2)
        el = e * l2
        sv = svec_ref[...]
        tv = tvec_ref[...]
        for k in range(NSLC):
            sv = sv + e[:, k * 128:(k + 1) * 128]
            tv = tv + el[:, k * 128:(k + 1) * 128]
        svec_ref[...] = sv
        tvec_ref[...] = tv

        @pl.when(j == NBLK - 1)
        def _fin():
            s = jnp.sum(svec_ref[...], axis=1, keepdims=True)
            t2 = jnp.sum(tvec_ref[...], axis=1, keepdims=True)
            logs = jnp.log(s)
            logz_ref[...] = m_ref[...] + logs
            ent_ref[...] = logs - t2 / s


@functools.partial(
    pl.kernel,
    mesh=plsc.VectorSubcoreMesh(core_axis_name="c", subcore_axis_name="s"),
    out_type=jax.ShapeDtypeStruct((B,), jnp.float32),
    scratch_types=[
        pltpu.VMEM((CH,), jnp.int32),
        pltpu.VMEM((NQ, 128), jnp.int32),
        pltpu.VMEM((NQ, 128), jnp.float32),
        pltpu.VMEM((CH,), jnp.float32),
        pltpu.VMEM((S, CH), jnp.float32),
        pltpu.VMEM((CH,), jnp.float32),
        pltpu.SemaphoreType.DMA,
    ],
)
def _sc_gather(wp_flat, bp_hbm, st3_hbm, act_hbm, la_hbm,
               a_v, idx_v, g_v, bpa_v, st_v, out_v, sem):
    wid = lax.axis_index("s") * NC + lax.axis_index("c")
    base = wid * CH
    pltpu.sync_copy(act_hbm.at[pl.ds(base, CH)], a_v)
    pltpu.sync_copy(st3_hbm.at[wid], st_v)
    pltpu.async_copy(bp_hbm.at[a_v], bpa_v, sem).wait()
    for k in range(S):
        for c in range(CH // 16):
            lin = k * CH + c * 16
            q, off = lin // 128, lin % 128
            av = a_v[pl.ds(c * 16, 16)]
            idx_v[q, pl.ds(off, 16)] = av + k * A
    copies = [
        pltpu.async_copy(wp_flat.at[idx_v.at[q]], g_v.at[q], sem)
        for q in range(NQ)
    ]
    for cp in copies:
        cp.wait()
    for c in range(CH // 16):
        acc = bpa_v[pl.ds(c * 16, 16)]
        for k in range(S):
            lin = k * CH + c * 16
            q, off = lin // 128, lin % 128
            acc = acc + g_v[q, pl.ds(off, 16)] * st_v[k, pl.ds(c * 16, 16)]
        out_v[pl.ds(c * 16, 16)] = acc
    pltpu.sync_copy(out_v, la_hbm.at[pl.ds(base, CH)])


@jax.jit
def _ac_call(state, action, Wp, bp, Wv, bv):
    bp_p = jnp.pad(bp, (0, NPAD - A), constant_values=-1e30)
    wpa = jnp.concatenate(
        [jnp.pad(Wp, ((0, 0), (0, NPAD - A))),
         bp_p[None, :],
         jnp.ones((1, NPAD), jnp.float32)], axis=0)
    state_aug = jnp.concatenate(
        [state, jnp.ones((B, 1), jnp.float32),
         jnp.zeros((B, 1), jnp.float32)], axis=1)
    wv_aug = jnp.concatenate(
        [Wv, bv[None, :], jnp.zeros((1, 1), jnp.float32)], axis=0)
    act_i = action.astype(jnp.int32)
    wp_flat = Wp.reshape(S * A)
    st3 = state.T.reshape(S, NW, CH).transpose(1, 0, 2)

    la = _sc_gather(wp_flat, bp, st3, act_i)

    value, logz, ent = pl.pallas_call(
        _ac_body,
        grid=(2, NBLK),
        in_specs=[
            pl.BlockSpec((B, S + 2), lambda p, j: (0, 0)),
            pl.BlockSpec((S + 2, ABLK), lambda p, j: (0, j)),
            pl.BlockSpec((S + 2, 1), lambda p, j: (0, 0)),
        ],
        out_specs=[
            pl.BlockSpec((B, 1), lambda p, j: (0, 0)),
            pl.BlockSpec((B, 1), lambda p, j: (0, 0)),
            pl.BlockSpec((B, 1), lambda p, j: (0, 0)),
        ],
        out_shape=[
            jax.ShapeDtypeStruct((B, 1), jnp.float32),
            jax.ShapeDtypeStruct((B, 1), jnp.float32),
            jax.ShapeDtypeStruct((B, 1), jnp.float32),
        ],
        scratch_shapes=[
            pltpu.VMEM((8, 128), jnp.float32),
            pltpu.VMEM((B, 1), jnp.float32),
            pltpu.VMEM((B, 128), jnp.float32),
            pltpu.VMEM((B, 128), jnp.float32),
        ],
    )(state_aug, wpa, wv_aug)
    return value, la - logz[:, 0], ent[:, 0]


def kernel(state, action, Wp, bp, Wv, bv):
    return _ac_call(state, action, Wp, bp, Wv, bv)
